# baseline (speedup 1.0000x reference)
import functools

import jax
import jax.numpy as jnp
from jax import lax
from jax.experimental import pallas as pl
from jax.experimental.pallas import tpu as pltpu
from jax.experimental.pallas import tpu_sc as plsc

N = 10000
E = 320000
D = 128
F1, F2, F3 = 128, 64, 32

NSUB = 16
NCORE = 2
NW = NSUB * NCORE
C = 128
CHUNKS = 80
E_PAD = NW * CHUNKS * C
N_PAD = 10112
RPS = N_PAD // NSUB
DEG_W = 16

_mesh = plsc.VectorSubcoreMesh(core_axis_name="c", subcore_axis_name="s",
                               num_cores=NCORE, num_subcores=NSUB)
_UNTILED = pltpu.CompilerParams(use_tc_tiling_on_sc=False)
_f32 = jnp.float32


def _sc_degree(dst3, zeros_deg):
    @functools.partial(
        pl.kernel,
        out_type=jax.ShapeDtypeStruct((NCORE, N_PAD, DEG_W), _f32),
        mesh=_mesh,
        compiler_params=_UNTILED,
        scratch_types=[
            pltpu.VMEM((CHUNKS, C), jnp.int32),
            pltpu.VMEM((C, DEG_W), _f32),
            pltpu.VMEM_SHARED((N_PAD, DEG_W), _f32),
        ],
    )
    def k(dst_hbm, zeros_hbm, out_hbm, dst_all, ones_v, acc_sh):
        cid = lax.axis_index("c")
        sid = lax.axis_index("s")
        wid = cid * NSUB + sid
        r0 = sid * RPS
        acc_rows = pl.ds(r0, RPS)

        @pl.loop(0, C)
        def _(i):
            ones_v[i] = jnp.full((DEG_W,), 1.0, _f32)

        pltpu.sync_copy(dst_hbm.at[wid], dst_all)
        pltpu.sync_copy(zeros_hbm.at[acc_rows], acc_sh.at[acc_rows])
        plsc.subcore_barrier()

        @pl.loop(0, CHUNKS)
        def _(j):
            pltpu.sync_copy(ones_v, acc_sh.at[dst_all.at[j]], add=True)

        plsc.subcore_barrier()
        pltpu.sync_copy(acc_sh.at[acc_rows], out_hbm.at[cid, acc_rows])

    return k(dst3, zeros_deg)


def _sc_aggregate(src3, dst3, table, zeros_nf, feat):
    nbuf = 2 if feat > 64 else 4

    @functools.partial(
        pl.kernel,
        out_type=jax.ShapeDtypeStruct((NCORE, N_PAD, feat), _f32),
        mesh=_mesh,
        compiler_params=_UNTILED,
        scratch_types=[
            pltpu.VMEM((CHUNKS, C), jnp.int32),
            [pltpu.VMEM((1, C), jnp.int32) for _ in range(nbuf)],
            [pltpu.VMEM((C, feat), _f32) for _ in range(nbuf)],
            pltpu.VMEM_SHARED((N_PAD, feat), _f32),
            [pltpu.SemaphoreType.DMA for _ in range(nbuf)],
            [pltpu.SemaphoreType.DMA for _ in range(nbuf)],
        ],
    )
    def k(src_hbm, dst_hbm, tab_hbm, zeros_hbm, out_hbm,
          src_all, dst_v, rows_v, acc_sh, sem_g, sem_s):
        cid = lax.axis_index("c")
        sid = lax.axis_index("s")
        wid = cid * NSUB + sid
        r0 = sid * RPS
        acc_rows = pl.ds(r0, RPS)

        pltpu.sync_copy(src_hbm.at[wid], src_all)
        pltpu.sync_copy(zeros_hbm.at[acc_rows], acc_sh.at[acc_rows])
        plsc.subcore_barrier()

        for b in range(nbuf - 1):
            pltpu.make_async_copy(tab_hbm.at[src_all.at[b]], rows_v[b], sem_g[b]).start()

        @pl.loop(0, CHUNKS, step=nbuf)
        def _(j):
            for b in range(nbuf):
                jj = j + b
                ahead = (b + nbuf - 1) % nbuf

                pltpu.make_async_copy(
                    tab_hbm.at[src_all.at[jj]], rows_v[b], sem_g[b]).wait()
                pltpu.sync_copy(dst_hbm.at[wid, pl.ds(jj, 1)], dst_v[b])
                pltpu.async_copy(rows_v[b], acc_sh.at[dst_v[b].at[0]],
                                 sem_s[b], add=True)

                @pl.when(jj + nbuf - 1 < CHUNKS)
                def _():
                    @pl.when(jj >= 1)
                    def _():
                        pltpu.make_async_copy(
                            rows_v[ahead], acc_sh.at[dst_v[ahead].at[0]],
                            sem_s[ahead]).wait()

                    pltpu.make_async_copy(
                        tab_hbm.at[src_all.at[jj + nbuf - 1]],
                        rows_v[ahead], sem_g[ahead]).start()

        for b in range(nbuf):
            pltpu.make_async_copy(rows_v[b], acc_sh.at[dst_v[b].at[0]],
                                  sem_s[b]).wait()

        plsc.subcore_barrier()
        pltpu.sync_copy(acc_sh.at[acc_rows], out_hbm.at[cid, acc_rows])

    return k(src3, dst3, table, zeros_nf)


def _dot(a, b):
    return lax.dot_general(a, b, (((1,), (0,)), ((), ())),
                           precision=lax.Precision.HIGHEST,
                           preferred_element_type=_f32)


def _tc_matmul(x, w):
    def body(x_ref, w_ref, o_ref):
        o_ref[...] = _dot(x_ref[...], w_ref[...])

    return pl.pallas_call(
        body,
        out_shape=jax.ShapeDtypeStruct((x.shape[0], w.shape[1]), _f32),
    )(x, w)


def _tc_scale(degp, hw):
    R = 2000

    def body(d_ref, h_ref, dinv_ref, dgi_ref, g_ref, s_ref):
        deg = d_ref[0, :, 0:1] + d_ref[1, :, 0:1] + 1.0
        dinv = lax.rsqrt(deg)
        dgi = 1.0 / deg
        dinv_ref[...] = dinv
        dgi_ref[...] = dgi
        h = h_ref[...]
        g_ref[...] = h * dinv
        s_ref[...] = h * dgi

    return pl.pallas_call(
        body,
        grid=(N // R,),
        in_specs=[
            pl.BlockSpec((2, R, DEG_W), lambda i: (0, i, 0)),
            pl.BlockSpec((R, F1), lambda i: (i, 0)),
        ],
        out_specs=[
            pl.BlockSpec((R, 1), lambda i: (i, 0)),
            pl.BlockSpec((R, 1), lambda i: (i, 0)),
            pl.BlockSpec((R, F1), lambda i: (i, 0)),
            pl.BlockSpec((R, F1), lambda i: (i, 0)),
        ],
        out_shape=[
            jax.ShapeDtypeStruct((N, 1), _f32),
            jax.ShapeDtypeStruct((N, 1), _f32),
            jax.ShapeDtypeStruct((N, F1), _f32),
            jax.ShapeDtypeStruct((N, F1), _f32),
        ],
    )(degp, hw)


def _tc_mid(accp, selfk, dinv, dgi, b_row, w_next, fi, fo):
    R = 2000

    def body(a_ref, s_ref, di_ref, dg_ref, b_ref, w_ref, g_ref, so_ref):
        acc = a_ref[0, :, :] + a_ref[1, :, :]
        h = di_ref[...] * acc + s_ref[...] + b_ref[...]
        h = jnp.maximum(h, 0.0)
        hw = _dot(h, w_ref[...])
        g_ref[...] = hw * di_ref[...]
        so_ref[...] = hw * dg_ref[...]

    return pl.pallas_call(
        body,
        grid=(N // R,),
        in_specs=[
            pl.BlockSpec((2, R, fi), lambda i: (0, i, 0)),
            pl.BlockSpec((R, fi), lambda i: (i, 0)),
            pl.BlockSpec((R, 1), lambda i: (i, 0)),
            pl.BlockSpec((R, 1), lambda i: (i, 0)),
            pl.BlockSpec((1, fi), lambda i: (0, 0)),
            pl.BlockSpec((fi, fo), lambda i: (0, 0)),
        ],
        out_specs=[
            pl.BlockSpec((R, fo), lambda i: (i, 0)),
            pl.BlockSpec((R, fo), lambda i: (i, 0)),
        ],
        out_shape=[
            jax.ShapeDtypeStruct((N, fo), _f32),
            jax.ShapeDtypeStruct((N, fo), _f32),
        ],
    )(accp, selfk, dinv, dgi, b_row, w_next)


def _tc_final(accp, selfk, dinv, b_row, wa):
    def body(a_ref, s_ref, di_ref, b_ref, wa_ref, o_ref):
        acc = a_ref[0, :N, :F3] + a_ref[1, :N, :F3]
        h = di_ref[...] * acc + s_ref[...] + b_ref[...]
        colmean = jnp.sum(h, axis=0, keepdims=True) * (1.0 / N)
        ctx = jnp.tanh(_dot(colmean, wa_ref[...]))
        logits = lax.dot_general(h, ctx, (((1,), (1,)), ((), ())),
                                 precision=lax.Precision.HIGHEST,
                                 preferred_element_type=_f32)
        sig = 1.0 / (1.0 + jnp.exp(-logits))
        o_ref[...] = lax.dot_general(h, sig, (((0,), (0,)), ((), ())),
                                     precision=lax.Precision.HIGHEST,
                                     preferred_element_type=_f32)

    return pl.pallas_call(
        body,
        out_shape=jax.ShapeDtypeStruct((F3, 1), _f32),
    )(accp, selfk, dinv, b_row, wa)


def _embed(ei, x, consts):
    w1, b1, w2, b2, w3, b3, wa, zdeg, zf1, zf2, zf3 = consts
    src = ei[0].astype(jnp.int32)
    dst = ei[1].astype(jnp.int32)
    pad = E_PAD - E
    src3 = jnp.concatenate([src, jnp.zeros((pad,), jnp.int32)]).reshape(NW, CHUNKS, C)
    dst3 = jnp.concatenate([dst, jnp.full((pad,), N, jnp.int32)]).reshape(NW, CHUNKS, C)

    degp = _sc_degree(dst3, zdeg)
    hw1 = _tc_matmul(x, w1)
    dinv, dgi, g1, s1 = _tc_scale(degp, hw1)
    acc1 = _sc_aggregate(src3, dst3, g1, zf1, F1)
    g2, s2 = _tc_mid(acc1, s1, dinv, dgi, b1.reshape(1, F1), w2, F1, F2)
    acc2 = _sc_aggregate(src3, dst3, g2, zf2, F2)
    g3, s3 = _tc_mid(acc2, s2, dinv, dgi, b2.reshape(1, F2), w3, F2, F3)
    acc3 = _sc_aggregate(src3, dst3, g3, zf3, F3)
    return _tc_final(acc3, s3, dinv, b3.reshape(1, F3), wa)


def kernel(edge_index_1, edge_index_2, features_1, features_2,
           W1, b1, W2, b2, W3, b3, Wa):
    consts = (
        W1, b1, W2, b2, W3, b3, Wa,
        jnp.zeros((N_PAD, DEG_W), _f32),
        jnp.zeros((N_PAD, F1), _f32),
        jnp.zeros((N_PAD, F2), _f32),
        jnp.zeros((N_PAD, F3), _f32),
    )
    p1 = _embed(edge_index_1, features_1, consts)
    p2 = _embed(edge_index_2, features_2, consts)
    return (p1, p2)


# speedup vs baseline: 1.0442x; 1.0442x over previous
import functools

import jax
import jax.numpy as jnp
from jax import lax
from jax.experimental import pallas as pl
from jax.experimental.pallas import tpu as pltpu
from jax.experimental.pallas import tpu_sc as plsc

N = 10000
E = 320000
D = 128
F1, F2, F3 = 128, 64, 32

NSUB = 16
NCORE = 2
NW = NSUB * NCORE
C = 128
CHUNKS = 80
E_PAD = NW * CHUNKS * C
N_PAD = 10112
RPS = N_PAD // NSUB
DEG_W = 16

_mesh = plsc.VectorSubcoreMesh(core_axis_name="c", subcore_axis_name="s",
                               num_cores=NCORE, num_subcores=NSUB)
_UNTILED = pltpu.CompilerParams(use_tc_tiling_on_sc=False)
_f32 = jnp.float32


def _sc_degree(dst3, zeros_deg):
    @functools.partial(
        pl.kernel,
        out_type=jax.ShapeDtypeStruct((NCORE, N_PAD, DEG_W), _f32),
        mesh=_mesh,
        compiler_params=_UNTILED,
        scratch_types=[
            pltpu.VMEM((CHUNKS, C), jnp.int32),
            pltpu.VMEM((C, DEG_W), _f32),
            pltpu.VMEM_SHARED((N_PAD, DEG_W), _f32),
        ],
    )
    def k(dst_hbm, zeros_hbm, out_hbm, dst_all, ones_v, acc_sh):
        cid = lax.axis_index("c")
        sid = lax.axis_index("s")
        wid = cid * NSUB + sid
        r0 = sid * RPS
        acc_rows = pl.ds(r0, RPS)

        @pl.loop(0, C)
        def _(i):
            ones_v[i] = jnp.full((DEG_W,), 1.0, _f32)

        pltpu.sync_copy(dst_hbm.at[wid], dst_all)
        pltpu.sync_copy(zeros_hbm.at[acc_rows], acc_sh.at[acc_rows])
        plsc.subcore_barrier()

        @pl.loop(0, CHUNKS)
        def _(j):
            pltpu.sync_copy(ones_v, acc_sh.at[dst_all.at[j]], add=True)

        plsc.subcore_barrier()
        pltpu.sync_copy(acc_sh.at[acc_rows], out_hbm.at[cid, acc_rows])

    return k(dst3, zeros_deg)


def _sc_aggregate(src3, dst3, table, zeros_nf, feat):
    nbuf = 2 if feat > 64 else 4

    @functools.partial(
        pl.kernel,
        out_type=jax.ShapeDtypeStruct((NCORE, N_PAD, feat), _f32),
        mesh=_mesh,
        compiler_params=_UNTILED,
        scratch_types=[
            pltpu.VMEM((CHUNKS, C), jnp.int32),
            [pltpu.VMEM((1, C), jnp.int32) for _ in range(nbuf)],
            [pltpu.VMEM((C, feat), _f32) for _ in range(nbuf)],
            pltpu.VMEM_SHARED((N_PAD, feat), _f32),
            [pltpu.SemaphoreType.DMA for _ in range(nbuf)],
        ],
    )
    def k(src_hbm, dst_hbm, tab_hbm, zeros_hbm, out_hbm,
          src_all, dst_v, rows_v, acc_sh, sem_g):
        cid = lax.axis_index("c")
        sid = lax.axis_index("s")
        wid = cid * NSUB + sid
        r0 = sid * RPS
        acc_rows = pl.ds(r0, RPS)

        pltpu.sync_copy(src_hbm.at[wid], src_all)
        pltpu.sync_copy(zeros_hbm.at[acc_rows], acc_sh.at[acc_rows])
        plsc.subcore_barrier()

        for b in range(nbuf - 1):
            pltpu.make_async_copy(tab_hbm.at[src_all.at[b]], rows_v[b], sem_g[b]).start()

        @pl.loop(0, CHUNKS, step=nbuf)
        def _(j):
            for b in range(nbuf):
                jj = j + b
                ahead = (b + nbuf - 1) % nbuf

                @pl.when(jj + nbuf - 1 < CHUNKS)
                def _():
                    pltpu.make_async_copy(
                        tab_hbm.at[src_all.at[jj + nbuf - 1]],
                        rows_v[ahead], sem_g[ahead]).start()

                pltpu.sync_copy(dst_hbm.at[wid, pl.ds(jj, 1)], dst_v[b])
                pltpu.make_async_copy(
                    tab_hbm.at[src_all.at[jj]], rows_v[b], sem_g[b]).wait()
                pltpu.sync_copy(rows_v[b], acc_sh.at[dst_v[b].at[0]], add=True)

        plsc.subcore_barrier()
        pltpu.sync_copy(acc_sh.at[acc_rows], out_hbm.at[cid, acc_rows])

    return k(src3, dst3, table, zeros_nf)


def _dot(a, b):
    return lax.dot_general(a, b, (((1,), (0,)), ((), ())),
                           precision=lax.Precision.HIGHEST,
                           preferred_element_type=_f32)


def _tc_matmul(x, w):
    def body(x_ref, w_ref, o_ref):
        o_ref[...] = _dot(x_ref[...], w_ref[...])

    return pl.pallas_call(
        body,
        out_shape=jax.ShapeDtypeStruct((x.shape[0], w.shape[1]), _f32),
    )(x, w)


def _tc_scale(degp, hw):
    R = 2000

    def body(d_ref, h_ref, dinv_ref, dgi_ref, g_ref, s_ref):
        deg = d_ref[0, :, 0:1] + d_ref[1, :, 0:1] + 1.0
        dinv = lax.rsqrt(deg)
        dgi = 1.0 / deg
        dinv_ref[...] = dinv
        dgi_ref[...] = dgi
        h = h_ref[...]
        g_ref[...] = h * dinv
        s_ref[...] = h * dgi

    return pl.pallas_call(
        body,
        grid=(N // R,),
        in_specs=[
            pl.BlockSpec((2, R, DEG_W), lambda i: (0, i, 0)),
            pl.BlockSpec((R, F1), lambda i: (i, 0)),
        ],
        out_specs=[
            pl.BlockSpec((R, 1), lambda i: (i, 0)),
            pl.BlockSpec((R, 1), lambda i: (i, 0)),
            pl.BlockSpec((R, F1), lambda i: (i, 0)),
            pl.BlockSpec((R, F1), lambda i: (i, 0)),
        ],
        out_shape=[
            jax.ShapeDtypeStruct((N, 1), _f32),
            jax.ShapeDtypeStruct((N, 1), _f32),
            jax.ShapeDtypeStruct((N, F1), _f32),
            jax.ShapeDtypeStruct((N, F1), _f32),
        ],
    )(degp, hw)


def _tc_mid(accp, selfk, dinv, dgi, b_row, w_next, fi, fo):
    R = 2000

    def body(a_ref, s_ref, di_ref, dg_ref, b_ref, w_ref, g_ref, so_ref):
        acc = a_ref[0, :, :] + a_ref[1, :, :]
        h = di_ref[...] * acc + s_ref[...] + b_ref[...]
        h = jnp.maximum(h, 0.0)
        hw = _dot(h, w_ref[...])
        g_ref[...] = hw * di_ref[...]
        so_ref[...] = hw * dg_ref[...]

    return pl.pallas_call(
        body,
        grid=(N // R,),
        in_specs=[
            pl.BlockSpec((2, R, fi), lambda i: (0, i, 0)),
            pl.BlockSpec((R, fi), lambda i: (i, 0)),
            pl.BlockSpec((R, 1), lambda i: (i, 0)),
            pl.BlockSpec((R, 1), lambda i: (i, 0)),
            pl.BlockSpec((1, fi), lambda i: (0, 0)),
            pl.BlockSpec((fi, fo), lambda i: (0, 0)),
        ],
        out_specs=[
            pl.BlockSpec((R, fo), lambda i: (i, 0)),
            pl.BlockSpec((R, fo), lambda i: (i, 0)),
        ],
        out_shape=[
            jax.ShapeDtypeStruct((N, fo), _f32),
            jax.ShapeDtypeStruct((N, fo), _f32),
        ],
    )(accp, selfk, dinv, dgi, b_row, w_next)


def _tc_final(accp, selfk, dinv, b_row, wa):
    def body(a_ref, s_ref, di_ref, b_ref, wa_ref, o_ref):
        acc = a_ref[0, :N, :F3] + a_ref[1, :N, :F3]
        h = di_ref[...] * acc + s_ref[...] + b_ref[...]
        colmean = jnp.sum(h, axis=0, keepdims=True) * (1.0 / N)
        ctx = jnp.tanh(_dot(colmean, wa_ref[...]))
        logits = lax.dot_general(h, ctx, (((1,), (1,)), ((), ())),
                                 precision=lax.Precision.HIGHEST,
                                 preferred_element_type=_f32)
        sig = 1.0 / (1.0 + jnp.exp(-logits))
        o_ref[...] = lax.dot_general(h, sig, (((0,), (0,)), ((), ())),
                                     precision=lax.Precision.HIGHEST,
                                     preferred_element_type=_f32)

    return pl.pallas_call(
        body,
        out_shape=jax.ShapeDtypeStruct((F3, 1), _f32),
    )(accp, selfk, dinv, b_row, wa)


def _embed(ei, x, consts):
    w1, b1, w2, b2, w3, b3, wa, zdeg, zf1, zf2, zf3 = consts
    src = ei[0].astype(jnp.int32)
    dst = ei[1].astype(jnp.int32)
    pad = E_PAD - E
    src3 = jnp.concatenate([src, jnp.zeros((pad,), jnp.int32)]).reshape(NW, CHUNKS, C)
    dst3 = jnp.concatenate([dst, jnp.full((pad,), N, jnp.int32)]).reshape(NW, CHUNKS, C)

    degp = _sc_degree(dst3, zdeg)
    hw1 = _tc_matmul(x, w1)
    dinv, dgi, g1, s1 = _tc_scale(degp, hw1)
    acc1 = _sc_aggregate(src3, dst3, g1, zf1, F1)
    g2, s2 = _tc_mid(acc1, s1, dinv, dgi, b1.reshape(1, F1), w2, F1, F2)
    acc2 = _sc_aggregate(src3, dst3, g2, zf2, F2)
    g3, s3 = _tc_mid(acc2, s2, dinv, dgi, b2.reshape(1, F2), w3, F2, F3)
    acc3 = _sc_aggregate(src3, dst3, g3, zf3, F3)
    return _tc_final(acc3, s3, dinv, b3.reshape(1, F3), wa)


def kernel(edge_index_1, edge_index_2, features_1, features_2,
           W1, b1, W2, b2, W3, b3, Wa):
    consts = (
        W1, b1, W2, b2, W3, b3, Wa,
        jnp.zeros((N_PAD, DEG_W), _f32),
        jnp.zeros((N_PAD, F1), _f32),
        jnp.zeros((N_PAD, F2), _f32),
        jnp.zeros((N_PAD, F3), _f32),
    )
    p1 = _embed(edge_index_1, features_1, consts)
    p2 = _embed(edge_index_2, features_2, consts)
    return (p1, p2)


# speedup vs baseline: 1.2022x; 1.1514x over previous
import functools

import jax
import jax.numpy as jnp
from jax import lax
from jax.experimental import pallas as pl
from jax.experimental.pallas import tpu as pltpu
from jax.experimental.pallas import tpu_sc as plsc

N = 10000
E = 320000
D = 128
F1, F2, F3 = 128, 64, 32

NSUB = 16
NCORE = 2
NW = NSUB * NCORE
C = 128
CHUNKS = 80
E_PAD = NW * CHUNKS * C
N_PAD = 10112
RPS = N_PAD // NSUB
DEG_W = 16

_mesh = plsc.VectorSubcoreMesh(core_axis_name="c", subcore_axis_name="s",
                               num_cores=NCORE, num_subcores=NSUB)
_UNTILED = pltpu.CompilerParams(use_tc_tiling_on_sc=False)
_f32 = jnp.float32


def _sc_degree(dst3, zeros_deg):
    @functools.partial(
        pl.kernel,
        out_type=jax.ShapeDtypeStruct((NCORE, N_PAD, DEG_W), _f32),
        mesh=_mesh,
        compiler_params=_UNTILED,
        scratch_types=[
            pltpu.VMEM((CHUNKS, C), jnp.int32),
            pltpu.VMEM((C, DEG_W), _f32),
            pltpu.VMEM_SHARED((N_PAD, DEG_W), _f32),
        ],
    )
    def k(dst_hbm, zeros_hbm, out_hbm, dst_all, ones_v, acc_sh):
        cid = lax.axis_index("c")
        sid = lax.axis_index("s")
        wid = cid * NSUB + sid
        r0 = sid * RPS
        acc_rows = pl.ds(r0, RPS)

        @pl.loop(0, C)
        def _(i):
            ones_v[i] = jnp.full((DEG_W,), 1.0, _f32)

        pltpu.sync_copy(dst_hbm.at[wid], dst_all)
        pltpu.sync_copy(zeros_hbm.at[acc_rows], acc_sh.at[acc_rows])
        plsc.subcore_barrier()

        @pl.loop(0, CHUNKS)
        def _(j):
            pltpu.sync_copy(ones_v, acc_sh.at[dst_all.at[j]], add=True)

        plsc.subcore_barrier()
        pltpu.sync_copy(acc_sh.at[acc_rows], out_hbm.at[cid, acc_rows])

    return k(dst3, zeros_deg)


def _sc_aggregate(src3, dst3, table, zeros_nf, feat):
    nbuf = 2 if feat > 64 else 4

    @functools.partial(
        pl.kernel,
        out_type=jax.ShapeDtypeStruct((NCORE, N_PAD, feat), _f32),
        mesh=_mesh,
        compiler_params=_UNTILED,
        scratch_types=[
            pltpu.VMEM((CHUNKS, C), jnp.int32),
            [pltpu.VMEM((1, C), jnp.int32) for _ in range(nbuf)],
            [pltpu.VMEM((C, feat), _f32) for _ in range(nbuf)],
            pltpu.VMEM_SHARED((N_PAD, feat), _f32),
            [pltpu.SemaphoreType.DMA for _ in range(nbuf)],
        ],
    )
    def k(src_hbm, dst_hbm, tab_hbm, zeros_hbm, out_hbm,
          src_all, dst_v, rows_v, acc_sh, sem_g):
        cid = lax.axis_index("c")
        sid = lax.axis_index("s")
        wid = cid * NSUB + sid
        r0 = sid * RPS
        acc_rows = pl.ds(r0, RPS)

        pltpu.sync_copy(src_hbm.at[wid], src_all)
        pltpu.sync_copy(zeros_hbm.at[acc_rows], acc_sh.at[acc_rows])
        plsc.subcore_barrier()

        for b in range(nbuf - 1):
            pltpu.make_async_copy(tab_hbm.at[src_all.at[b]], rows_v[b], sem_g[b]).start()

        @pl.loop(0, CHUNKS, step=nbuf)
        def _(j):
            for b in range(nbuf):
                jj = j + b
                ahead = (b + nbuf - 1) % nbuf

                @pl.when(jj + nbuf - 1 < CHUNKS)
                def _():
                    pltpu.make_async_copy(
                        tab_hbm.at[src_all.at[jj + nbuf - 1]],
                        rows_v[ahead], sem_g[ahead]).start()

                pltpu.sync_copy(dst_hbm.at[wid, pl.ds(jj, 1)], dst_v[b])
                pltpu.make_async_copy(
                    tab_hbm.at[src_all.at[jj]], rows_v[b], sem_g[b]).wait()
                pltpu.sync_copy(rows_v[b], acc_sh.at[dst_v[b].at[0]], add=True)

        plsc.subcore_barrier()
        pltpu.sync_copy(acc_sh.at[acc_rows], out_hbm.at[cid, acc_rows])

    return k(src3, dst3, table, zeros_nf)


def _dot(a, b):
    return lax.dot_general(a, b, (((1,), (0,)), ((), ())),
                           precision=lax.Precision.HIGHEST,
                           preferred_element_type=_f32)


def _tc_matmul(x, w):
    def body(x_ref, w_ref, o_ref):
        o_ref[...] = _dot(x_ref[...], w_ref[...])

    return pl.pallas_call(
        body,
        out_shape=jax.ShapeDtypeStruct((x.shape[0], w.shape[1]), _f32),
    )(x, w)


def _tc_scale(degp, hw):
    R = 2000

    def body(d_ref, h_ref, dinv_ref, dgi_ref, g_ref, s_ref):
        deg = d_ref[0, :, 0:1] + d_ref[1, :, 0:1] + 1.0
        dinv = lax.rsqrt(deg)
        dgi = 1.0 / deg
        dinv_ref[...] = dinv
        dgi_ref[...] = dgi
        h = h_ref[...]
        g_ref[...] = h * dinv
        s_ref[...] = h * dgi

    return pl.pallas_call(
        body,
        grid=(N // R,),
        in_specs=[
            pl.BlockSpec((2, R, DEG_W), lambda i: (0, i, 0)),
            pl.BlockSpec((R, F1), lambda i: (i, 0)),
        ],
        out_specs=[
            pl.BlockSpec((R, 1), lambda i: (i, 0)),
            pl.BlockSpec((R, 1), lambda i: (i, 0)),
            pl.BlockSpec((R, F1), lambda i: (i, 0)),
            pl.BlockSpec((R, F1), lambda i: (i, 0)),
        ],
        out_shape=[
            jax.ShapeDtypeStruct((N, 1), _f32),
            jax.ShapeDtypeStruct((N, 1), _f32),
            jax.ShapeDtypeStruct((N, F1), _f32),
            jax.ShapeDtypeStruct((N, F1), _f32),
        ],
    )(degp, hw)


def _tc_mid(accp, selfk, dinv, dgi, b_row, w_next, fi, fo):
    R = 2000

    def body(a_ref, s_ref, di_ref, dg_ref, b_ref, w_ref, g_ref, so_ref):
        acc = a_ref[0, :, :] + a_ref[1, :, :]
        h = di_ref[...] * acc + s_ref[...] + b_ref[...]
        h = jnp.maximum(h, 0.0)
        hw = _dot(h, w_ref[...])
        g_ref[...] = hw * di_ref[...]
        so_ref[...] = hw * dg_ref[...]

    return pl.pallas_call(
        body,
        grid=(N // R,),
        in_specs=[
            pl.BlockSpec((2, R, fi), lambda i: (0, i, 0)),
            pl.BlockSpec((R, fi), lambda i: (i, 0)),
            pl.BlockSpec((R, 1), lambda i: (i, 0)),
            pl.BlockSpec((R, 1), lambda i: (i, 0)),
            pl.BlockSpec((1, fi), lambda i: (0, 0)),
            pl.BlockSpec((fi, fo), lambda i: (0, 0)),
        ],
        out_specs=[
            pl.BlockSpec((R, fo), lambda i: (i, 0)),
            pl.BlockSpec((R, fo), lambda i: (i, 0)),
        ],
        out_shape=[
            jax.ShapeDtypeStruct((N, fo), _f32),
            jax.ShapeDtypeStruct((N, fo), _f32),
        ],
    )(accp, selfk, dinv, dgi, b_row, w_next)


def _tc_final_pair(acc_a, self_a, dinv_a, acc_b, self_b, dinv_b, b_row, wa):
    def one(a_ref, s_ref, di_ref, b_ref, wa_ref, o_ref):
        acc = a_ref[0, :N, :F3] + a_ref[1, :N, :F3]
        h = di_ref[...] * acc + s_ref[...] + b_ref[...]
        colmean = jnp.sum(h, axis=0, keepdims=True) * (1.0 / N)
        ctx = jnp.tanh(_dot(colmean, wa_ref[...]))
        logits = lax.dot_general(h, ctx, (((1,), (1,)), ((), ())),
                                 precision=lax.Precision.HIGHEST,
                                 preferred_element_type=_f32)
        sig = 1.0 / (1.0 + jnp.exp(-logits))
        o_ref[...] = lax.dot_general(h, sig, (((0,), (0,)), ((), ())),
                                     precision=lax.Precision.HIGHEST,
                                     preferred_element_type=_f32)

    def body(aa_ref, sa_ref, da_ref, ab_ref, sb_ref, db_ref, b_ref, wa_ref,
             oa_ref, ob_ref):
        one(aa_ref, sa_ref, da_ref, b_ref, wa_ref, oa_ref)
        one(ab_ref, sb_ref, db_ref, b_ref, wa_ref, ob_ref)

    return pl.pallas_call(
        body,
        out_shape=[
            jax.ShapeDtypeStruct((F3, 1), _f32),
            jax.ShapeDtypeStruct((F3, 1), _f32),
        ],
    )(acc_a, self_a, dinv_a, acc_b, self_b, dinv_b, b_row, wa)


def _embed(ei, x, consts):
    w1, b1, w2, b2, w3, b3, wa, zdeg, zf1, zf2, zf3 = consts
    src = ei[0].astype(jnp.int32)
    dst = ei[1].astype(jnp.int32)
    pad = E_PAD - E
    src3 = jnp.concatenate([src, jnp.zeros((pad,), jnp.int32)]).reshape(NW, CHUNKS, C)
    dst3 = jnp.concatenate([dst, jnp.full((pad,), N, jnp.int32)]).reshape(NW, CHUNKS, C)

    degp = _sc_degree(dst3, zdeg)
    hw1 = _tc_matmul(x, w1)
    dinv, dgi, g1, s1 = _tc_scale(degp, hw1)
    acc1 = _sc_aggregate(src3, dst3, g1, zf1, F1)
    g2, s2 = _tc_mid(acc1, s1, dinv, dgi, b1.reshape(1, F1), w2, F1, F2)
    acc2 = _sc_aggregate(src3, dst3, g2, zf2, F2)
    g3, s3 = _tc_mid(acc2, s2, dinv, dgi, b2.reshape(1, F2), w3, F2, F3)
    acc3 = _sc_aggregate(src3, dst3, g3, zf3, F3)
    return acc3, s3, dinv


def kernel(edge_index_1, edge_index_2, features_1, features_2,
           W1, b1, W2, b2, W3, b3, Wa):
    consts = (
        W1, b1, W2, b2, W3, b3, Wa,
        jnp.zeros((N_PAD, DEG_W), _f32),
        jnp.zeros((N_PAD, F1), _f32),
        jnp.zeros((N_PAD, F2), _f32),
        jnp.zeros((N_PAD, F3), _f32),
    )
    acc3_1, s3_1, dinv_1 = _embed(edge_index_1, features_1, consts)
    acc3_2, s3_2, dinv_2 = _embed(edge_index_2, features_2, consts)
    p1, p2 = _tc_final_pair(acc3_1, s3_1, dinv_1, acc3_2, s3_2, dinv_2,
                            b3.reshape(1, F3), Wa)
    return (p1, p2)
